# baseline (speedup 1.0000x reference)
import jax
import jax.numpy as jnp
from jax import lax
from jax.experimental import pallas as pl
from jax.experimental.pallas import tpu as pltpu
from jax.experimental.pallas import tpu_sc as plsc

NC = 2
NS = 16
BLK = 128
CB = 4
CHUNK = CB * BLK
SCALE = 1e4


def _sc_body(x01_hbm, src_hbm, dst_hbm, ea_hbm, out_hbm, tablev,
             srcv0, dstv0, eav0, vyv0, srcv1, dstv1, eav1, vyv1,
             acc0, acc1, acc2, acc3, sem_in0, sem_in1, sem_s0, sem_s1):
    n = x01_hbm.shape[0] // 2
    nchunks_total = src_hbm.shape[0] // CHUNK
    c = lax.axis_index("c")
    s = lax.axis_index("s")
    t = s % 2
    w = (s // 2) * NC + c
    bufs = ((srcv0, dstv0, eav0, vyv0), (srcv1, dstv1, eav1, vyv1))
    sems_in = (sem_in0, sem_in1)
    sems_s = (sem_s0, sem_s1)

    pltpu.sync_copy(x01_hbm.at[pl.ds(pl.multiple_of(t * n, 8), n)], tablev)

    zero_f = jnp.zeros((16,), jnp.float32)
    for g in range(2 * CHUNK // 16):
        eav0[pl.ds(g * 16, 16)] = zero_f

    stripe = (n // NS + 7) // 8 * 8
    last = n - stripe * (NS - 1)
    row0 = s * stripe
    ZC = 2 * CHUNK

    def _zero(nrows):
        for acc in (acc0, acc1, acc2, acc3):
            for j in range(nrows // ZC):
                pltpu.sync_copy(eav0, acc.at[pl.ds(row0 + j * ZC, ZC)])
            tail = nrows % ZC
            if tail:
                pltpu.sync_copy(
                    eav0.at[pl.ds(0, tail)],
                    acc.at[pl.ds(row0 + (nrows // ZC) * ZC, tail)])

    @pl.when(s < NS - 1)
    def _():
        _zero(stripe)

    @pl.when(s == NS - 1)
    def _():
        _zero(last)

    plsc.subcore_barrier()

    nw = NS * NC // 2
    base_cnt = nchunks_total // nw
    extra = nchunks_total - base_cnt * nw
    nch = jnp.where(w < extra, base_cnt + 1, base_cnt)
    start = w * base_cnt + jnp.minimum(w, extra)

    scale_f = zero_f + jnp.float32(SCALE)
    lanes2 = lax.iota(jnp.int32, 16) * 2

    def fire_ins(ch, p):
        srcv, dstv, eav, vyv = bufs[p]
        pltpu.async_copy(src_hbm.at[pl.ds(ch * CHUNK, CHUNK)], srcv,
                         sems_in[p])
        pltpu.async_copy(dst_hbm.at[pl.ds(ch * CB, CB)], dstv, sems_in[p])
        pltpu.async_copy(ea_hbm.at[pl.ds(ch * ZC, ZC)], eav, sems_in[p])

    def drain_ins(p):
        srcv, dstv, eav, vyv = bufs[p]
        pltpu.make_async_copy(src_hbm.at[pl.ds(0, CHUNK)], srcv,
                              sems_in[p]).wait()
        pltpu.make_async_copy(dst_hbm.at[pl.ds(0, CB)], dstv,
                              sems_in[p]).wait()
        pltpu.make_async_copy(ea_hbm.at[pl.ds(0, ZC)], eav,
                              sems_in[p]).wait()

    def drain_scat(p):
        srcv, dstv, eav, vyv = bufs[p]
        for q in range(CB):
            qs = pl.ds(q * BLK, BLK)
            pltpu.make_async_copy(eav.at[qs], acc0.at[dstv.at[q]],
                                  sems_s[p]).wait()
            pltpu.make_async_copy(vyv.at[qs], acc2.at[dstv.at[q]],
                                  sems_s[p]).wait()

    def compute_fire(p):
        srcv, dstv, eav, vyv = bufs[p]
        for q in range(CB):
            for gq in range(BLK // 16):
                g = q * (BLK // 16) + gq
                sl = pl.ds(g * 16, 16)
                gsl = pl.ds(gq * 16, 16)
                srcl = srcv[sl]
                dstl = dstv[q, gsl]
                xs = plsc.load_gather(tablev, [srcl])
                xd = plsc.load_gather(tablev, [dstl])
                diff = xs - xd
                ea_idx = lanes2 + (g * 32)
                e0 = plsc.load_gather(eav, [ea_idx])
                e1 = plsc.load_gather(eav, [ea_idx + 1])
                mx = jnp.where(e0 != 0.0, scale_f, zero_f)
                my = jnp.where(e1 != 0.0, scale_f, zero_f)
                eav[sl] = mx * diff
                vyv[sl] = my * diff
            qs = pl.ds(q * BLK, BLK)

            @pl.when(t == 0)
            def _():
                pltpu.async_copy(eav.at[qs], acc0.at[dstv.at[q]], sems_s[p],
                                 add=True)
                pltpu.async_copy(vyv.at[qs], acc2.at[dstv.at[q]], sems_s[p],
                                 add=True)

            @pl.when(t == 1)
            def _():
                pltpu.async_copy(eav.at[qs], acc1.at[dstv.at[q]], sems_s[p],
                                 add=True)
                pltpu.async_copy(vyv.at[qs], acc3.at[dstv.at[q]], sems_s[p],
                                 add=True)

    def step(i, p):
        @pl.when(i >= 1)
        def _():
            drain_scat(1 - p)

        @pl.when(i + 1 < nch)
        def _():
            fire_ins(start + i + 1, 1 - p)

        drain_ins(p)
        compute_fire(p)

    fire_ins(start, 0)

    def pair_body(j, carry):
        step(2 * j, 0)

        @pl.when(2 * j + 1 < nch)
        def _():
            step(2 * j + 1, 1)

        return carry

    lax.fori_loop(0, (nch + 1) // 2, pair_body, 0)

    @pl.when(nch % 2 == 1)
    def _():
        drain_scat(0)

    @pl.when(nch % 2 == 0)
    def _():
        drain_scat(1)

    plsc.subcore_barrier()

    def _writeout(nrows):
        for k, acc in enumerate((acc0, acc1, acc2, acc3)):
            out0 = c * 4 * n + k * n + row0
            for j in range(nrows // ZC):
                pltpu.sync_copy(acc.at[pl.ds(row0 + j * ZC, ZC)], eav0)
                pltpu.sync_copy(eav0, out_hbm.at[pl.ds(out0 + j * ZC, ZC)])
            tail = nrows % ZC
            if tail:
                jfull = (nrows // ZC) * ZC
                pltpu.sync_copy(acc.at[pl.ds(row0 + jfull, tail)],
                                eav0.at[pl.ds(0, tail)])
                pltpu.sync_copy(eav0.at[pl.ds(0, tail)],
                                out_hbm.at[pl.ds(out0 + jfull, tail)])

    @pl.when(s < NS - 1)
    def _():
        _writeout(stripe)

    @pl.when(s == NS - 1)
    def _():
        _writeout(last)


def _combine_body(p_ref, o_ref):
    o_ref[...] = p_ref[0] + p_ref[1]


def kernel(x, edge_index, edge_attr):
    n = x.shape[0]
    e = edge_index.shape[1]
    assert e % CHUNK == 0 and n % 8 == 0 and (4 * n) % 128 == 0

    x01 = x[:, :2].astype(jnp.float32).T.reshape(-1)
    src = edge_index[0].astype(jnp.int32)
    dst2 = edge_index[1].astype(jnp.int32).reshape(e // BLK, BLK)
    ea = edge_attr[:, :2].astype(jnp.float32).reshape(-1)

    mesh = plsc.VectorSubcoreMesh(core_axis_name="c", subcore_axis_name="s")
    sc_call = pl.kernel(
        _sc_body,
        out_type=jax.ShapeDtypeStruct((NC * 4 * n,), jnp.float32),
        mesh=mesh,
        compiler_params=pltpu.CompilerParams(needs_layout_passes=False),
        scratch_types=[
            pltpu.VMEM((n,), jnp.float32),
            pltpu.VMEM((CHUNK,), jnp.int32),
            pltpu.VMEM((CB, BLK), jnp.int32),
            pltpu.VMEM((2 * CHUNK,), jnp.float32),
            pltpu.VMEM((CHUNK,), jnp.float32),
            pltpu.VMEM((CHUNK,), jnp.int32),
            pltpu.VMEM((CB, BLK), jnp.int32),
            pltpu.VMEM((2 * CHUNK,), jnp.float32),
            pltpu.VMEM((CHUNK,), jnp.float32),
            pltpu.VMEM_SHARED((n,), jnp.float32),
            pltpu.VMEM_SHARED((n,), jnp.float32),
            pltpu.VMEM_SHARED((n,), jnp.float32),
            pltpu.VMEM_SHARED((n,), jnp.float32),
            pltpu.SemaphoreType.DMA,
            pltpu.SemaphoreType.DMA,
            pltpu.SemaphoreType.DMA,
            pltpu.SemaphoreType.DMA,
        ],
    )
    partials = sc_call(x01, src, dst2, ea)

    rows = (4 * n) // 128
    combined = pl.pallas_call(
        _combine_body,
        out_shape=jax.ShapeDtypeStruct((rows, 128), jnp.float32),
    )(partials.reshape(NC, rows, 128))

    r = combined.reshape(4, n)
    return (r[0], r[1], r[3], r[2])


# speedup vs baseline: 8.4091x; 8.4091x over previous
import jax
import jax.numpy as jnp
from jax import lax
from jax.experimental import pallas as pl
from jax.experimental.pallas import tpu as pltpu
from jax.experimental.pallas import tpu_sc as plsc

NC = 2
NS = 16
BLK = 128
CB = 4
CHUNK = CB * BLK
SCALE = 1e4


def _sc_body(x01_hbm, src_hbm, dst_hbm, ea0_hbm, ea1_hbm, out_hbm, tablev,
             srcv0, dstv0, vxv0, vyv0, srcv1, dstv1, vxv1, vyv1,
             acc0, acc1, acc2, acc3, sem_in0, sem_in1, sem_s0, sem_s1):
    n = x01_hbm.shape[0] // 2
    nchunks_total = src_hbm.shape[0] // CHUNK
    c = lax.axis_index("c")
    s = lax.axis_index("s")
    t = s % 2
    w = (s // 2) * NC + c
    bufs = ((srcv0, dstv0, vxv0, vyv0), (srcv1, dstv1, vxv1, vyv1))
    sems_in = (sem_in0, sem_in1)
    sems_s = (sem_s0, sem_s1)

    pltpu.sync_copy(x01_hbm.at[pl.ds(pl.multiple_of(t * n, 8), n)], tablev)

    zero_f = jnp.zeros((16,), jnp.float32)
    for g in range(CHUNK // 16):
        vxv0[pl.ds(g * 16, 16)] = zero_f

    stripe = (n // NS + 7) // 8 * 8
    last = n - stripe * (NS - 1)
    row0 = s * stripe

    def _zero(nrows):
        for acc in (acc0, acc1, acc2, acc3):
            for j in range(nrows // CHUNK):
                pltpu.sync_copy(vxv0, acc.at[pl.ds(row0 + j * CHUNK, CHUNK)])
            tail = nrows % CHUNK
            if tail:
                pltpu.sync_copy(
                    vxv0.at[pl.ds(0, tail)],
                    acc.at[pl.ds(row0 + (nrows // CHUNK) * CHUNK, tail)])

    @pl.when(s < NS - 1)
    def _():
        _zero(stripe)

    @pl.when(s == NS - 1)
    def _():
        _zero(last)

    plsc.subcore_barrier()

    nw = NS * NC // 2
    base_cnt = nchunks_total // nw
    extra = nchunks_total - base_cnt * nw
    nch = jnp.where(w < extra, base_cnt + 1, base_cnt)
    start = w * base_cnt + jnp.minimum(w, extra)

    scale_f = zero_f + jnp.float32(SCALE)

    def fire_ins(ch, p):
        srcv, dstv, vxv, vyv = bufs[p]
        off = ch * CHUNK
        pltpu.async_copy(src_hbm.at[pl.ds(off, CHUNK)], srcv, sems_in[p])
        pltpu.async_copy(dst_hbm.at[pl.ds(ch * CB, CB)], dstv, sems_in[p])
        pltpu.async_copy(ea0_hbm.at[pl.ds(off, CHUNK)], vxv, sems_in[p])
        pltpu.async_copy(ea1_hbm.at[pl.ds(off, CHUNK)], vyv, sems_in[p])

    def drain_ins(p):
        srcv, dstv, vxv, vyv = bufs[p]
        pltpu.make_async_copy(src_hbm.at[pl.ds(0, CHUNK)], srcv,
                              sems_in[p]).wait()
        pltpu.make_async_copy(dst_hbm.at[pl.ds(0, CB)], dstv,
                              sems_in[p]).wait()
        pltpu.make_async_copy(ea0_hbm.at[pl.ds(0, CHUNK)], vxv,
                              sems_in[p]).wait()
        pltpu.make_async_copy(ea1_hbm.at[pl.ds(0, CHUNK)], vyv,
                              sems_in[p]).wait()

    def drain_scat(p):
        srcv, dstv, vxv, vyv = bufs[p]
        for q in range(CB):
            qs = pl.ds(q * BLK, BLK)
            pltpu.make_async_copy(vxv.at[qs], acc0.at[dstv.at[q]],
                                  sems_s[p]).wait()
            pltpu.make_async_copy(vyv.at[qs], acc2.at[dstv.at[q]],
                                  sems_s[p]).wait()

    def compute_fire(p):
        srcv, dstv, vxv, vyv = bufs[p]
        for q in range(CB):
            for gq in range(BLK // 16):
                g = q * (BLK // 16) + gq
                sl = pl.ds(g * 16, 16)
                srcl = srcv[sl]
                dstl = dstv[q, pl.ds(gq * 16, 16)]
                xs = plsc.load_gather(tablev, [srcl])
                xd = plsc.load_gather(tablev, [dstl])
                diff = xs - xd
                mx = jnp.where(vxv[sl] != 0.0, scale_f, zero_f)
                my = jnp.where(vyv[sl] != 0.0, scale_f, zero_f)
                vxv[sl] = mx * diff
                vyv[sl] = my * diff
            qs = pl.ds(q * BLK, BLK)

            @pl.when(t == 0)
            def _():
                pltpu.async_copy(vxv.at[qs], acc0.at[dstv.at[q]], sems_s[p],
                                 add=True)
                pltpu.async_copy(vyv.at[qs], acc2.at[dstv.at[q]], sems_s[p],
                                 add=True)

            @pl.when(t == 1)
            def _():
                pltpu.async_copy(vxv.at[qs], acc1.at[dstv.at[q]], sems_s[p],
                                 add=True)
                pltpu.async_copy(vyv.at[qs], acc3.at[dstv.at[q]], sems_s[p],
                                 add=True)

    def step(i, p):
        @pl.when(i >= 1)
        def _():
            drain_scat(1 - p)

        @pl.when(i + 1 < nch)
        def _():
            fire_ins(start + i + 1, 1 - p)

        drain_ins(p)
        compute_fire(p)

    fire_ins(start, 0)

    def pair_body(j, carry):
        step(2 * j, 0)

        @pl.when(2 * j + 1 < nch)
        def _():
            step(2 * j + 1, 1)

        return carry

    lax.fori_loop(0, (nch + 1) // 2, pair_body, 0)

    @pl.when(nch % 2 == 1)
    def _():
        drain_scat(0)

    @pl.when(nch % 2 == 0)
    def _():
        drain_scat(1)

    plsc.subcore_barrier()

    def _writeout(nrows):
        for k, acc in enumerate((acc0, acc1, acc2, acc3)):
            out0 = c * 4 * n + k * n + row0
            for j in range(nrows // CHUNK):
                pltpu.sync_copy(acc.at[pl.ds(row0 + j * CHUNK, CHUNK)], vxv0)
                pltpu.sync_copy(vxv0, out_hbm.at[pl.ds(out0 + j * CHUNK,
                                                       CHUNK)])
            tail = nrows % CHUNK
            if tail:
                jfull = (nrows // CHUNK) * CHUNK
                pltpu.sync_copy(acc.at[pl.ds(row0 + jfull, tail)],
                                vxv0.at[pl.ds(0, tail)])
                pltpu.sync_copy(vxv0.at[pl.ds(0, tail)],
                                out_hbm.at[pl.ds(out0 + jfull, tail)])

    @pl.when(s < NS - 1)
    def _():
        _writeout(stripe)

    @pl.when(s == NS - 1)
    def _():
        _writeout(last)


def _combine_body(pa_ref, pb_ref, o_ref):
    o_ref[...] = (pa_ref[0] + pa_ref[1]) + (pb_ref[0] + pb_ref[1])


def kernel(x, edge_index, edge_attr):
    n = x.shape[0]
    e = edge_index.shape[1]
    e2 = e // 2
    assert e2 % CHUNK == 0 and (n * 4) % 128 == 0 and n % 8 == 0

    x01 = x[:, :2].astype(jnp.float32).T.reshape(-1)

    mesh = plsc.VectorSubcoreMesh(core_axis_name="c", subcore_axis_name="s")
    sc_call = pl.kernel(
        _sc_body,
        out_type=jax.ShapeDtypeStruct((NC * 4 * n,), jnp.float32),
        mesh=mesh,
        compiler_params=pltpu.CompilerParams(needs_layout_passes=False),
        scratch_types=[
            pltpu.VMEM((n,), jnp.float32),
            pltpu.VMEM((CHUNK,), jnp.int32),
            pltpu.VMEM((CB, BLK), jnp.int32),
            pltpu.VMEM((CHUNK,), jnp.float32),
            pltpu.VMEM((CHUNK,), jnp.float32),
            pltpu.VMEM((CHUNK,), jnp.int32),
            pltpu.VMEM((CB, BLK), jnp.int32),
            pltpu.VMEM((CHUNK,), jnp.float32),
            pltpu.VMEM((CHUNK,), jnp.float32),
            pltpu.VMEM_SHARED((n,), jnp.float32),
            pltpu.VMEM_SHARED((n,), jnp.float32),
            pltpu.VMEM_SHARED((n,), jnp.float32),
            pltpu.VMEM_SHARED((n,), jnp.float32),
            pltpu.SemaphoreType.DMA,
            pltpu.SemaphoreType.DMA,
            pltpu.SemaphoreType.DMA,
            pltpu.SemaphoreType.DMA,
        ],
    )
    parts = []
    for lo in (0, e2):
        src = edge_index[0, lo:lo + e2].astype(jnp.int32)
        dst2 = edge_index[1, lo:lo + e2].astype(jnp.int32).reshape(
            e2 // BLK, BLK)
        ea0 = edge_attr[lo:lo + e2, 0].astype(jnp.float32)
        ea1 = edge_attr[lo:lo + e2, 1].astype(jnp.float32)
        parts.append(sc_call(x01, src, dst2, ea0, ea1))

    rows = (4 * n) // 128
    combined = pl.pallas_call(
        _combine_body,
        out_shape=jax.ShapeDtypeStruct((rows, 128), jnp.float32),
    )(parts[0].reshape(NC, rows, 128), parts[1].reshape(NC, rows, 128))

    r = combined.reshape(4, n)
    return (r[0], r[1], r[3], r[2])


# speedup vs baseline: 11.4126x; 1.3572x over previous
import jax
import jax.numpy as jnp
from jax import lax
from jax.experimental import pallas as pl
from jax.experimental.pallas import tpu as pltpu
from jax.experimental.pallas import tpu_sc as plsc

NC = 2
NS = 16
BLK = 128
CB = 4
CHUNK = CB * BLK
SCALE = 1e4


def _sc_body(x01_hbm, src_hbm, dst_hbm, ea0_hbm, ea1_hbm, out_hbm, tablev,
             srcv0, dstv0, vxv0, vyv0, srcv1, dstv1, vxv1, vyv1,
             acc0, acc1, acc2, acc3, sem_in0, sem_in1, sem_s0, sem_s1):
    n = x01_hbm.shape[0] // 2
    nchunks_total = src_hbm.shape[0] // CHUNK
    c = lax.axis_index("c")
    s = lax.axis_index("s")
    t = s % 2
    w = (s // 2) * NC + c
    bufs = ((srcv0, dstv0, vxv0, vyv0), (srcv1, dstv1, vxv1, vyv1))
    sems_in = (sem_in0, sem_in1)
    sems_s = (sem_s0, sem_s1)

    pltpu.sync_copy(x01_hbm.at[pl.ds(pl.multiple_of(t * n, 8), n)], tablev)

    zero_f = jnp.zeros((16,), jnp.float32)
    for g in range(CHUNK // 16):
        vxv0[pl.ds(g * 16, 16)] = zero_f

    stripe = (n // NS + 7) // 8 * 8
    last = n - stripe * (NS - 1)
    row0 = s * stripe

    def _zero(nrows):
        descs = []
        for acc in (acc0, acc1, acc2, acc3):
            for j in range(nrows // CHUNK):
                descs.append(pltpu.async_copy(
                    vxv0, acc.at[pl.ds(row0 + j * CHUNK, CHUNK)], sem_in0))
            tail = nrows % CHUNK
            if tail:
                descs.append(pltpu.async_copy(
                    vxv0.at[pl.ds(0, tail)],
                    acc.at[pl.ds(row0 + (nrows // CHUNK) * CHUNK, tail)],
                    sem_in0))
        for d in descs:
            d.wait()

    @pl.when(s < NS - 1)
    def _():
        _zero(stripe)

    @pl.when(s == NS - 1)
    def _():
        _zero(last)

    plsc.subcore_barrier()

    nw = NS * NC // 2
    base_cnt = nchunks_total // nw
    extra = nchunks_total - base_cnt * nw
    nch = jnp.where(w < extra, base_cnt + 1, base_cnt)
    start = w * base_cnt + jnp.minimum(w, extra)

    scale_f = zero_f + jnp.float32(SCALE)

    def fire_ins(ch, p):
        srcv, dstv, vxv, vyv = bufs[p]
        off = ch * CHUNK
        pltpu.async_copy(src_hbm.at[pl.ds(off, CHUNK)], srcv, sems_in[p])
        pltpu.async_copy(dst_hbm.at[pl.ds(ch * CB, CB)], dstv, sems_in[p])
        pltpu.async_copy(ea0_hbm.at[pl.ds(off, CHUNK)], vxv, sems_in[p])
        pltpu.async_copy(ea1_hbm.at[pl.ds(off, CHUNK)], vyv, sems_in[p])

    def drain_ins(p):
        srcv, dstv, vxv, vyv = bufs[p]
        pltpu.make_async_copy(src_hbm.at[pl.ds(0, CHUNK)], srcv,
                              sems_in[p]).wait()
        pltpu.make_async_copy(dst_hbm.at[pl.ds(0, CB)], dstv,
                              sems_in[p]).wait()
        pltpu.make_async_copy(ea0_hbm.at[pl.ds(0, CHUNK)], vxv,
                              sems_in[p]).wait()
        pltpu.make_async_copy(ea1_hbm.at[pl.ds(0, CHUNK)], vyv,
                              sems_in[p]).wait()

    def drain_scat(p):
        srcv, dstv, vxv, vyv = bufs[p]
        for q in range(CB):
            qs = pl.ds(q * BLK, BLK)
            pltpu.make_async_copy(vxv.at[qs], acc0.at[dstv.at[q]],
                                  sems_s[p]).wait()
            pltpu.make_async_copy(vyv.at[qs], acc2.at[dstv.at[q]],
                                  sems_s[p]).wait()

    def compute_fire(p):
        srcv, dstv, vxv, vyv = bufs[p]
        for q in range(CB):
            for gq in range(BLK // 16):
                g = q * (BLK // 16) + gq
                sl = pl.ds(g * 16, 16)
                srcl = srcv[sl]
                dstl = dstv[q, pl.ds(gq * 16, 16)]
                xs = plsc.load_gather(tablev, [srcl])
                xd = plsc.load_gather(tablev, [dstl])
                diff = xs - xd
                mx = jnp.where(vxv[sl] != 0.0, scale_f, zero_f)
                my = jnp.where(vyv[sl] != 0.0, scale_f, zero_f)
                vxv[sl] = mx * diff
                vyv[sl] = my * diff
            qs = pl.ds(q * BLK, BLK)

            @pl.when(t == 0)
            def _():
                pltpu.async_copy(vxv.at[qs], acc0.at[dstv.at[q]], sems_s[p],
                                 add=True)
                pltpu.async_copy(vyv.at[qs], acc2.at[dstv.at[q]], sems_s[p],
                                 add=True)

            @pl.when(t == 1)
            def _():
                pltpu.async_copy(vxv.at[qs], acc1.at[dstv.at[q]], sems_s[p],
                                 add=True)
                pltpu.async_copy(vyv.at[qs], acc3.at[dstv.at[q]], sems_s[p],
                                 add=True)

    def step(i, p):
        @pl.when(i >= 1)
        def _():
            drain_scat(1 - p)

        @pl.when(i + 1 < nch)
        def _():
            fire_ins(start + i + 1, 1 - p)

        drain_ins(p)
        compute_fire(p)

    fire_ins(start, 0)

    def pair_body(j, carry):
        step(2 * j, 0)

        @pl.when(2 * j + 1 < nch)
        def _():
            step(2 * j + 1, 1)

        return carry

    lax.fori_loop(0, (nch + 1) // 2, pair_body, 0)

    @pl.when(nch % 2 == 1)
    def _():
        drain_scat(0)

    @pl.when(nch % 2 == 0)
    def _():
        drain_scat(1)

    plsc.subcore_barrier()

    def _writeout(nrows):
        descs = []
        for k, acc in enumerate((acc0, acc1, acc2, acc3)):
            descs.append(pltpu.async_copy(
                acc.at[pl.ds(row0, nrows)],
                tablev.at[pl.ds(k * stripe, nrows)], sem_in0))
        for d in descs:
            d.wait()
        descs = []
        for k in range(4):
            out0 = c * 4 * n + k * n + row0
            descs.append(pltpu.async_copy(
                tablev.at[pl.ds(k * stripe, nrows)],
                out_hbm.at[pl.ds(out0, nrows)], sem_in0))
        for d in descs:
            d.wait()

    @pl.when(s < NS - 1)
    def _():
        _writeout(stripe)

    @pl.when(s == NS - 1)
    def _():
        _writeout(last)


def _combine_body(p_ref, o_ref):
    o_ref[...] = p_ref[0] + p_ref[1]


def kernel(x, edge_index, edge_attr):
    n = x.shape[0]
    e = edge_index.shape[1]
    assert e % CHUNK == 0 and (n * 4) % 128 == 0 and n % 8 == 0

    x01 = x[:, :2].astype(jnp.float32).T.reshape(-1)
    src = edge_index[0].astype(jnp.int32)
    dst2 = edge_index[1].astype(jnp.int32).reshape(e // BLK, BLK)
    ea0 = edge_attr[:, 0].astype(jnp.float32)
    ea1 = edge_attr[:, 1].astype(jnp.float32)

    mesh = plsc.VectorSubcoreMesh(core_axis_name="c", subcore_axis_name="s")
    sc_call = pl.kernel(
        _sc_body,
        out_type=jax.ShapeDtypeStruct((NC * 4 * n,), jnp.float32),
        mesh=mesh,
        compiler_params=pltpu.CompilerParams(needs_layout_passes=False),
        scratch_types=[
            pltpu.VMEM((n,), jnp.float32),
            pltpu.VMEM((CHUNK,), jnp.int32),
            pltpu.VMEM((CB, BLK), jnp.int32),
            pltpu.VMEM((CHUNK,), jnp.float32),
            pltpu.VMEM((CHUNK,), jnp.float32),
            pltpu.VMEM((CHUNK,), jnp.int32),
            pltpu.VMEM((CB, BLK), jnp.int32),
            pltpu.VMEM((CHUNK,), jnp.float32),
            pltpu.VMEM((CHUNK,), jnp.float32),
            pltpu.VMEM_SHARED((n,), jnp.float32),
            pltpu.VMEM_SHARED((n,), jnp.float32),
            pltpu.VMEM_SHARED((n,), jnp.float32),
            pltpu.VMEM_SHARED((n,), jnp.float32),
            pltpu.SemaphoreType.DMA,
            pltpu.SemaphoreType.DMA,
            pltpu.SemaphoreType.DMA,
            pltpu.SemaphoreType.DMA,
        ],
    )
    partials = sc_call(x01, src, dst2, ea0, ea1)

    rows = (4 * n) // 128
    combined = pl.pallas_call(
        _combine_body,
        out_shape=jax.ShapeDtypeStruct((rows, 128), jnp.float32),
    )(partials.reshape(NC, rows, 128))

    r = combined.reshape(4, n)
    return (r[0], r[1], r[3], r[2])


# speedup vs baseline: 11.8080x; 1.0346x over previous
import jax
import jax.numpy as jnp
from jax import lax
from jax.experimental import pallas as pl
from jax.experimental.pallas import tpu as pltpu
from jax.experimental.pallas import tpu_sc as plsc

NC = 2
NS = 16
BLK = 128
CB = 4
CHUNK = CB * BLK
SCALE = 1e4


def _sc_body(x01_hbm, ei_hbm, dst_hbm, ea0_hbm, ea1_hbm, out_hbm, tablev,
             srcv0, dstv0, vxv0, vyv0, srcv1, dstv1, vxv1, vyv1,
             acc0, acc1, acc2, acc3, sem_in0, sem_in1, sem_s0, sem_s1):
    n = x01_hbm.shape[0] // 2
    nchunks_total = ei_hbm.shape[1] // CHUNK
    c = lax.axis_index("c")
    s = lax.axis_index("s")
    t = s % 2
    w = (s // 2) * NC + c
    bufs = ((srcv0, dstv0, vxv0, vyv0), (srcv1, dstv1, vxv1, vyv1))
    sems_in = (sem_in0, sem_in1)
    sems_s = (sem_s0, sem_s1)

    pltpu.sync_copy(x01_hbm.at[pl.ds(pl.multiple_of(t * n, 8), n)], tablev)

    zero_f = jnp.zeros((16,), jnp.float32)
    for g in range(CHUNK // 16):
        vxv0[pl.ds(g * 16, 16)] = zero_f

    stripe = (n // NS + 7) // 8 * 8
    last = n - stripe * (NS - 1)
    row0 = s * stripe

    def _zero(nrows):
        descs = []
        for acc in (acc0, acc1, acc2, acc3):
            for j in range(nrows // CHUNK):
                descs.append(pltpu.async_copy(
                    vxv0, acc.at[pl.ds(row0 + j * CHUNK, CHUNK)], sem_in0))
            tail = nrows % CHUNK
            if tail:
                descs.append(pltpu.async_copy(
                    vxv0.at[pl.ds(0, tail)],
                    acc.at[pl.ds(row0 + (nrows // CHUNK) * CHUNK, tail)],
                    sem_in0))
        for d in descs:
            d.wait()

    @pl.when(s < NS - 1)
    def _():
        _zero(stripe)

    @pl.when(s == NS - 1)
    def _():
        _zero(last)

    plsc.subcore_barrier()

    nw = NS * NC // 2
    base_cnt = nchunks_total // nw
    extra = nchunks_total - base_cnt * nw
    nch = jnp.where(w < extra, base_cnt + 1, base_cnt)
    start = w * base_cnt + jnp.minimum(w, extra)

    scale_f = zero_f + jnp.float32(SCALE)

    def fire_ins(ch, p):
        srcv, dstv, vxv, vyv = bufs[p]
        off = ch * CHUNK
        pltpu.async_copy(ei_hbm.at[0, pl.ds(off, CHUNK)], srcv, sems_in[p])
        pltpu.async_copy(dst_hbm.at[pl.ds(ch * CB, CB)], dstv, sems_in[p])
        pltpu.async_copy(ea0_hbm.at[pl.ds(off, CHUNK)], vxv, sems_in[p])
        pltpu.async_copy(ea1_hbm.at[pl.ds(off, CHUNK)], vyv, sems_in[p])

    def drain_ins(p):
        srcv, dstv, vxv, vyv = bufs[p]
        pltpu.make_async_copy(ei_hbm.at[0, pl.ds(0, CHUNK)], srcv,
                              sems_in[p]).wait()
        pltpu.make_async_copy(dst_hbm.at[pl.ds(0, CB)], dstv,
                              sems_in[p]).wait()
        pltpu.make_async_copy(ea0_hbm.at[pl.ds(0, CHUNK)], vxv,
                              sems_in[p]).wait()
        pltpu.make_async_copy(ea1_hbm.at[pl.ds(0, CHUNK)], vyv,
                              sems_in[p]).wait()

    def drain_scat(p):
        srcv, dstv, vxv, vyv = bufs[p]
        for q in range(CB):
            qs = pl.ds(q * BLK, BLK)
            pltpu.make_async_copy(vxv.at[qs], acc0.at[dstv.at[q]],
                                  sems_s[p]).wait()
            pltpu.make_async_copy(vyv.at[qs], acc2.at[dstv.at[q]],
                                  sems_s[p]).wait()

    def compute_fire(p):
        srcv, dstv, vxv, vyv = bufs[p]
        for q in range(CB):
            for gq in range(BLK // 16):
                g = q * (BLK // 16) + gq
                sl = pl.ds(g * 16, 16)
                srcl = srcv[sl]
                dstl = dstv[q, pl.ds(gq * 16, 16)]
                xs = plsc.load_gather(tablev, [srcl])
                xd = plsc.load_gather(tablev, [dstl])
                diff = xs - xd
                mx = jnp.where(vxv[sl] != 0.0, scale_f, zero_f)
                my = jnp.where(vyv[sl] != 0.0, scale_f, zero_f)
                vxv[sl] = mx * diff
                vyv[sl] = my * diff
            qs = pl.ds(q * BLK, BLK)

            @pl.when(t == 0)
            def _():
                pltpu.async_copy(vxv.at[qs], acc0.at[dstv.at[q]], sems_s[p],
                                 add=True)
                pltpu.async_copy(vyv.at[qs], acc2.at[dstv.at[q]], sems_s[p],
                                 add=True)

            @pl.when(t == 1)
            def _():
                pltpu.async_copy(vxv.at[qs], acc1.at[dstv.at[q]], sems_s[p],
                                 add=True)
                pltpu.async_copy(vyv.at[qs], acc3.at[dstv.at[q]], sems_s[p],
                                 add=True)

    def step(i, p):
        @pl.when(i >= 1)
        def _():
            drain_scat(1 - p)

        @pl.when(i + 1 < nch)
        def _():
            fire_ins(start + i + 1, 1 - p)

        drain_ins(p)
        compute_fire(p)

    fire_ins(start, 0)

    def pair_body(j, carry):
        step(2 * j, 0)

        @pl.when(2 * j + 1 < nch)
        def _():
            step(2 * j + 1, 1)

        return carry

    lax.fori_loop(0, (nch + 1) // 2, pair_body, 0)

    @pl.when(nch % 2 == 1)
    def _():
        drain_scat(0)

    @pl.when(nch % 2 == 0)
    def _():
        drain_scat(1)

    plsc.subcore_barrier()

    def _writeout(nrows):
        descs = []
        for k, acc in enumerate((acc0, acc1, acc2, acc3)):
            descs.append(pltpu.async_copy(
                acc.at[pl.ds(row0, nrows)],
                tablev.at[pl.ds(k * stripe, nrows)], sem_in0))
        for d in descs:
            d.wait()
        descs = []
        for k in range(4):
            out0 = c * 4 * n + k * n + row0
            descs.append(pltpu.async_copy(
                tablev.at[pl.ds(k * stripe, nrows)],
                out_hbm.at[pl.ds(out0, nrows)], sem_in0))
        for d in descs:
            d.wait()

    @pl.when(s < NS - 1)
    def _():
        _writeout(stripe)

    @pl.when(s == NS - 1)
    def _():
        _writeout(last)


def _combine_body(p_ref, o_ref):
    o_ref[...] = p_ref[0] + p_ref[1]


def kernel(x, edge_index, edge_attr):
    n = x.shape[0]
    e = edge_index.shape[1]
    assert e % CHUNK == 0 and (n * 4) % 128 == 0 and n % 8 == 0

    x01 = x[:, :2].astype(jnp.float32).T.reshape(-1)
    ei = edge_index.astype(jnp.int32)
    dst2 = edge_index[1].astype(jnp.int32).reshape(e // BLK, BLK)
    ea0 = edge_attr[:, 0].astype(jnp.float32)
    ea1 = edge_attr[:, 1].astype(jnp.float32)

    mesh = plsc.VectorSubcoreMesh(core_axis_name="c", subcore_axis_name="s")
    sc_call = pl.kernel(
        _sc_body,
        out_type=jax.ShapeDtypeStruct((NC * 4 * n,), jnp.float32),
        mesh=mesh,
        compiler_params=pltpu.CompilerParams(needs_layout_passes=False),
        scratch_types=[
            pltpu.VMEM((n,), jnp.float32),
            pltpu.VMEM((CHUNK,), jnp.int32),
            pltpu.VMEM((CB, BLK), jnp.int32),
            pltpu.VMEM((CHUNK,), jnp.float32),
            pltpu.VMEM((CHUNK,), jnp.float32),
            pltpu.VMEM((CHUNK,), jnp.int32),
            pltpu.VMEM((CB, BLK), jnp.int32),
            pltpu.VMEM((CHUNK,), jnp.float32),
            pltpu.VMEM((CHUNK,), jnp.float32),
            pltpu.VMEM_SHARED((n,), jnp.float32),
            pltpu.VMEM_SHARED((n,), jnp.float32),
            pltpu.VMEM_SHARED((n,), jnp.float32),
            pltpu.VMEM_SHARED((n,), jnp.float32),
            pltpu.SemaphoreType.DMA,
            pltpu.SemaphoreType.DMA,
            pltpu.SemaphoreType.DMA,
            pltpu.SemaphoreType.DMA,
        ],
    )
    partials = sc_call(x01, ei, dst2, ea0, ea1)

    rows = (4 * n) // 128
    combined = pl.pallas_call(
        _combine_body,
        out_shape=jax.ShapeDtypeStruct((rows, 128), jnp.float32),
    )(partials.reshape(NC, rows, 128))

    r = combined.reshape(4, n)
    return (r[0], r[1], r[3], r[2])


# speedup vs baseline: 12.3494x; 1.0459x over previous
import jax
import jax.numpy as jnp
from jax import lax
from jax.experimental import pallas as pl
from jax.experimental.pallas import tpu as pltpu
from jax.experimental.pallas import tpu_sc as plsc

NC = 2
NS = 16
BLK = 128
CB = 4
CHUNK = CB * BLK
SCALE = 1e4


def _sc_body(x01_hbm, ei_hbm, ea0_hbm, ea1_hbm, out_hbm, tablev,
             srcv0, dstv0, vxv0, vyv0, srcv1, dstv1, vxv1, vyv1,
             acc0, acc1, acc2, acc3, sem_in0, sem_in1, sem_s0, sem_s1):
    n = x01_hbm.shape[0] // 2
    nchunks_total = ei_hbm.shape[1] // CHUNK
    c = lax.axis_index("c")
    s = lax.axis_index("s")
    t = s % 2
    w = (s // 2) * NC + c
    bufs = ((srcv0, dstv0, vxv0, vyv0), (srcv1, dstv1, vxv1, vyv1))
    sems_in = (sem_in0, sem_in1)
    sems_s = (sem_s0, sem_s1)

    pltpu.sync_copy(x01_hbm.at[pl.ds(pl.multiple_of(t * n, 8), n)], tablev)

    zero_f = jnp.zeros((16,), jnp.float32)
    for g in range(CHUNK // 16):
        vxv0[pl.ds(g * 16, 16)] = zero_f

    stripe = (n // NS + 7) // 8 * 8
    last = n - stripe * (NS - 1)
    row0 = s * stripe

    def _zero(nrows):
        descs = []
        for acc in (acc0, acc1, acc2, acc3):
            for j in range(nrows // CHUNK):
                descs.append(pltpu.async_copy(
                    vxv0, acc.at[pl.ds(row0 + j * CHUNK, CHUNK)], sem_in0))
            tail = nrows % CHUNK
            if tail:
                descs.append(pltpu.async_copy(
                    vxv0.at[pl.ds(0, tail)],
                    acc.at[pl.ds(row0 + (nrows // CHUNK) * CHUNK, tail)],
                    sem_in0))
        for d in descs:
            d.wait()

    @pl.when(s < NS - 1)
    def _():
        _zero(stripe)

    @pl.when(s == NS - 1)
    def _():
        _zero(last)

    plsc.subcore_barrier()

    nw = NS * NC // 2
    base_cnt = nchunks_total // nw
    extra = nchunks_total - base_cnt * nw
    nch = jnp.where(w < extra, base_cnt + 1, base_cnt)
    start = w * base_cnt + jnp.minimum(w, extra)

    scale_f = zero_f + jnp.float32(SCALE)

    def fire_ins(ch, p):
        srcv, dstv, vxv, vyv = bufs[p]
        off = ch * CHUNK
        pltpu.async_copy(ei_hbm.at[0, pl.ds(off, CHUNK)], srcv, sems_in[p])
        for q in range(CB):
            pltpu.async_copy(ei_hbm.at[1, pl.ds(off + q * BLK, BLK)],
                             dstv.at[q], sems_in[p])
        pltpu.async_copy(ea0_hbm.at[pl.ds(off, CHUNK)], vxv, sems_in[p])
        pltpu.async_copy(ea1_hbm.at[pl.ds(off, CHUNK)], vyv, sems_in[p])

    def drain_ins(p):
        srcv, dstv, vxv, vyv = bufs[p]
        pltpu.make_async_copy(ei_hbm.at[0, pl.ds(0, CHUNK)], srcv,
                              sems_in[p]).wait()
        for q in range(CB):
            pltpu.make_async_copy(ei_hbm.at[1, pl.ds(0, BLK)], dstv.at[q],
                                  sems_in[p]).wait()
        pltpu.make_async_copy(ea0_hbm.at[pl.ds(0, CHUNK)], vxv,
                              sems_in[p]).wait()
        pltpu.make_async_copy(ea1_hbm.at[pl.ds(0, CHUNK)], vyv,
                              sems_in[p]).wait()

    def drain_scat(p):
        srcv, dstv, vxv, vyv = bufs[p]
        for q in range(CB):
            qs = pl.ds(q * BLK, BLK)
            pltpu.make_async_copy(vxv.at[qs], acc0.at[dstv.at[q]],
                                  sems_s[p]).wait()
            pltpu.make_async_copy(vyv.at[qs], acc2.at[dstv.at[q]],
                                  sems_s[p]).wait()

    def compute_fire(p):
        srcv, dstv, vxv, vyv = bufs[p]
        for q in range(CB):
            for gq in range(BLK // 16):
                g = q * (BLK // 16) + gq
                sl = pl.ds(g * 16, 16)
                srcl = srcv[sl]
                dstl = dstv[q, pl.ds(gq * 16, 16)]
                xs = plsc.load_gather(tablev, [srcl])
                xd = plsc.load_gather(tablev, [dstl])
                diff = xs - xd
                mx = jnp.where(vxv[sl] != 0.0, scale_f, zero_f)
                my = jnp.where(vyv[sl] != 0.0, scale_f, zero_f)
                vxv[sl] = mx * diff
                vyv[sl] = my * diff
            qs = pl.ds(q * BLK, BLK)

            @pl.when(t == 0)
            def _():
                pltpu.async_copy(vxv.at[qs], acc0.at[dstv.at[q]], sems_s[p],
                                 add=True)
                pltpu.async_copy(vyv.at[qs], acc2.at[dstv.at[q]], sems_s[p],
                                 add=True)

            @pl.when(t == 1)
            def _():
                pltpu.async_copy(vxv.at[qs], acc1.at[dstv.at[q]], sems_s[p],
                                 add=True)
                pltpu.async_copy(vyv.at[qs], acc3.at[dstv.at[q]], sems_s[p],
                                 add=True)

    def step(i, p):
        @pl.when(i >= 1)
        def _():
            drain_scat(1 - p)

        @pl.when(i + 1 < nch)
        def _():
            fire_ins(start + i + 1, 1 - p)

        drain_ins(p)
        compute_fire(p)

    fire_ins(start, 0)

    def pair_body(j, carry):
        step(2 * j, 0)

        @pl.when(2 * j + 1 < nch)
        def _():
            step(2 * j + 1, 1)

        return carry

    lax.fori_loop(0, (nch + 1) // 2, pair_body, 0)

    @pl.when(nch % 2 == 1)
    def _():
        drain_scat(0)

    @pl.when(nch % 2 == 0)
    def _():
        drain_scat(1)

    plsc.subcore_barrier()

    def _writeout(nrows):
        descs = []
        for k, acc in enumerate((acc0, acc1, acc2, acc3)):
            descs.append(pltpu.async_copy(
                acc.at[pl.ds(row0, nrows)],
                tablev.at[pl.ds(k * stripe, nrows)], sem_in0))
        for d in descs:
            d.wait()
        descs = []
        for k in range(4):
            out0 = c * 4 * n + k * n + row0
            descs.append(pltpu.async_copy(
                tablev.at[pl.ds(k * stripe, nrows)],
                out_hbm.at[pl.ds(out0, nrows)], sem_in0))
        for d in descs:
            d.wait()

    @pl.when(s < NS - 1)
    def _():
        _writeout(stripe)

    @pl.when(s == NS - 1)
    def _():
        _writeout(last)


def _combine_body(p_ref, o_ref):
    o_ref[...] = p_ref[0] + p_ref[1]


def kernel(x, edge_index, edge_attr):
    n = x.shape[0]
    e = edge_index.shape[1]
    assert e % CHUNK == 0 and (n * 4) % 128 == 0 and n % 8 == 0

    x01 = x[:, :2].astype(jnp.float32).T.reshape(-1)
    ei = edge_index.astype(jnp.int32)
    ea0 = edge_attr[:, 0].astype(jnp.float32)
    ea1 = edge_attr[:, 1].astype(jnp.float32)

    mesh = plsc.VectorSubcoreMesh(core_axis_name="c", subcore_axis_name="s")
    sc_call = pl.kernel(
        _sc_body,
        out_type=jax.ShapeDtypeStruct((NC * 4 * n,), jnp.float32),
        mesh=mesh,
        compiler_params=pltpu.CompilerParams(needs_layout_passes=False),
        scratch_types=[
            pltpu.VMEM((n,), jnp.float32),
            pltpu.VMEM((CHUNK,), jnp.int32),
            pltpu.VMEM((CB, BLK), jnp.int32),
            pltpu.VMEM((CHUNK,), jnp.float32),
            pltpu.VMEM((CHUNK,), jnp.float32),
            pltpu.VMEM((CHUNK,), jnp.int32),
            pltpu.VMEM((CB, BLK), jnp.int32),
            pltpu.VMEM((CHUNK,), jnp.float32),
            pltpu.VMEM((CHUNK,), jnp.float32),
            pltpu.VMEM_SHARED((n,), jnp.float32),
            pltpu.VMEM_SHARED((n,), jnp.float32),
            pltpu.VMEM_SHARED((n,), jnp.float32),
            pltpu.VMEM_SHARED((n,), jnp.float32),
            pltpu.SemaphoreType.DMA,
            pltpu.SemaphoreType.DMA,
            pltpu.SemaphoreType.DMA,
            pltpu.SemaphoreType.DMA,
        ],
    )
    partials = sc_call(x01, ei, ea0, ea1)

    rows = (4 * n) // 128
    combined = pl.pallas_call(
        _combine_body,
        out_shape=jax.ShapeDtypeStruct((rows, 128), jnp.float32),
    )(partials.reshape(NC, rows, 128))

    r = combined.reshape(4, n)
    return (r[0], r[1], r[3], r[2])
